# baseline (speedup 1.0000x reference)
import functools
import math

import jax
import jax.numpy as jnp
from jax import lax
from jax.experimental import pallas as pl
from jax.experimental.pallas import tpu as pltpu
from jax.experimental.pallas import tpu_sc as plsc

_VOCAB = 100000
_EMBED = 64
_BATCH = 16384
_LMIN = math.log(0.1)
_LMAX = math.log(10.0)
_MUC = math.sqrt(2.0)

_NC = 2
_NS = 16
_NW = _NC * _NS
_BPW = _BATCH // _NW
_CH = 128
_NCH = _BPW // _CH
_U = 4

_mesh = plsc.VectorSubcoreMesh(core_axis_name="c", subcore_axis_name="s")


@functools.partial(
    pl.kernel,
    out_type=jax.ShapeDtypeStruct((_NC, 16), jnp.float32),
    mesh=_mesh,
    compiler_params=pltpu.CompilerParams(use_tc_tiling_on_sc=False),
    scratch_types=[
        pltpu.VMEM((_CH,), jnp.int32),
        pltpu.VMEM((_CH,), jnp.int32),
        pltpu.VMEM((_CH,), jnp.int32),
        pltpu.VMEM((_CH, _EMBED), jnp.float32),
        pltpu.VMEM((_CH, _EMBED), jnp.float32),
        pltpu.VMEM((_CH, _EMBED), jnp.float32),
        pltpu.VMEM((_CH, _EMBED), jnp.float32),
        pltpu.VMEM((_CH, _EMBED), jnp.float32),
        pltpu.VMEM((_CH, _EMBED), jnp.float32),
        pltpu.VMEM((16,), jnp.float32),
        pltpu.VMEM((4, 16), jnp.float32),
        pltpu.VMEM((16, 16), jnp.float32),
        pltpu.VMEM_SHARED((16, 16), jnp.float32),
        pltpu.SemaphoreType.DMA,
    ],
)
def _gauss_loss_sc(wi_hbm, wj_hbm, wn_hbm, mu_hbm, mup_hbm, mun_hbm,
                   ls_hbm, lsp_hbm, lsn_hbm, out_hbm,
                   ii_v, ij_v, in_v, rmi, rmj, rmn, rsi, rsj, rsn,
                   acc_v, svec_v, gath_v, shared, sem):
    cidx = lax.axis_index("c")
    sidx = lax.axis_index("s")
    wid = sidx * _NC + cidx
    base = wid * _BPW

    lane = lax.iota(jnp.int32, 16)
    zeros16 = jnp.zeros((16,), jnp.float32)

    def clip_mu(x):
        return jnp.minimum(jnp.maximum(x, -_MUC), _MUC)

    def clip_ls(x):
        return jnp.minimum(jnp.maximum(x, _LMIN), _LMAX)

    acc = jnp.float32(0.0)
    for ch in range(_NCH):
        cb = base + ch * _CH
        pltpu.sync_copy(wi_hbm.at[pl.ds(cb, _CH)], ii_v)
        pltpu.sync_copy(wj_hbm.at[pl.ds(cb, _CH)], ij_v)
        pltpu.sync_copy(wn_hbm.at[pl.ds(cb, _CH)], in_v)
        cps = [
            pltpu.async_copy(mu_hbm.at[ii_v], rmi, sem),
            pltpu.async_copy(mup_hbm.at[ij_v], rmj, sem),
            pltpu.async_copy(mun_hbm.at[in_v], rmn, sem),
            pltpu.async_copy(ls_hbm.at[ii_v], rsi, sem),
            pltpu.async_copy(lsp_hbm.at[ij_v], rsj, sem),
            pltpu.async_copy(lsn_hbm.at[in_v], rsn, sem),
        ]
        for cp in cps:
            cp.wait()

        def grp_body(g, acc):
            e0 = g * _U
            for u in range(_U):
                e = e0 + u
                vsum = zeros16
                for k in range(_EMBED // 16):
                    sl = pl.ds(k * 16, 16)
                    mi = clip_mu(rmi[e, sl])
                    mj = clip_mu(rmj[e, sl])
                    mn = clip_mu(rmn[e, sl])
                    li = clip_ls(rsi[e, sl])
                    lj = clip_ls(rsj[e, sl])
                    ln = clip_ls(rsn[e, sl])
                    inv_si = jnp.exp(-li)
                    dj = mi - mj
                    dn = mi - mn
                    num = jnp.exp(lj) - jnp.exp(ln) + dj * dj - dn * dn
                    vsum = vsum + num * inv_si + lj - ln
                s = [vsum[i] for i in range(16)]
                while len(s) > 1:
                    s = [s[2 * i] + s[2 * i + 1] for i in range(len(s) // 2)]
                acc = acc + jnp.maximum(1.0 + 0.5 * s[0], 0.0)
            return acc

        acc = lax.fori_loop(0, _CH // _U, grp_body, acc)

    acc_v[...] = jnp.where(lane == 0, acc, 0.0)
    pltpu.sync_copy(acc_v, shared.at[sidx])
    plsc.subcore_barrier()

    @pl.when(sidx == 0)
    def _():
        pltpu.sync_copy(shared, gath_v)
        tot = zeros16
        for r in range(16):
            tot = tot + gath_v[r, :]
        acc_v[...] = tot
        pltpu.sync_copy(acc_v, out_hbm.at[cidx])


def kernel(words_i, words_j, words_n, mu, mu_pos, mu_neg,
           log_sigma, log_sigma_pos, log_sigma_neg):
    partials = _gauss_loss_sc(
        words_i.astype(jnp.int32), words_j.astype(jnp.int32),
        words_n.astype(jnp.int32), mu, mu_pos, mu_neg,
        log_sigma, log_sigma_pos, log_sigma_neg)
    return jnp.sum(partials) * (1.0 / _BATCH)
